# Initial kernel scaffold; baseline (speedup 1.0000x reference)
import jax
import jax.numpy as jnp
from jax.experimental import pallas as pl


def kernel(inputs, emb):
    raise NotImplementedError("write your pallas kernel here")


# speedup vs baseline: 1.1865x; 1.1865x over previous
import functools

import jax
import jax.numpy as jnp
from jax import lax
from jax.experimental import pallas as pl
from jax.experimental.pallas import tpu as pltpu
from jax.experimental.pallas import tpu_sc as plsc

SIZE = 1000000
DIM = 16
BATCH = 16384
NCOLS = 51
NNEG = NCOLS - 1
EPS = 1e-10

NC = 2
NS = 16
NW = NC * NS

U_PER_W = BATCH // NW
V_PER_W = BATCH * NNEG // NW
V_CHUNK = 1024
V_STEPS = V_PER_W // V_CHUNK


def _sc_gather(emb, idx_u, idx_v):
    mesh = plsc.VectorSubcoreMesh(core_axis_name="c", subcore_axis_name="s")

    @functools.partial(
        pl.kernel,
        out_type=(
            jax.ShapeDtypeStruct((BATCH, DIM), jnp.float32),
            jax.ShapeDtypeStruct((BATCH * NNEG, DIM), jnp.float32),
        ),
        mesh=mesh,
        compiler_params=pltpu.CompilerParams(use_tc_tiling_on_sc=False),
        scratch_types=(
            pltpu.VMEM((U_PER_W,), jnp.int32),
            pltpu.VMEM((U_PER_W, DIM), jnp.float32),
            pltpu.VMEM((V_CHUNK,), jnp.int32),
            pltpu.VMEM((V_CHUNK, DIM), jnp.float32),
            pltpu.SemaphoreType.DMA,
        ),
    )
    def gather_kernel(emb_h, idxu_h, idxv_h, from_h, to_h,
                      idxu_buf, urow_buf, idxv_buf, vrow_buf, sem):
        wid = lax.axis_index("s") * NC + lax.axis_index("c")
        ubase = wid * U_PER_W
        pltpu.sync_copy(idxu_h.at[pl.ds(ubase, U_PER_W)], idxu_buf)
        pltpu.async_copy(emb_h.at[idxu_buf], urow_buf, sem).wait()
        pltpu.sync_copy(urow_buf, from_h.at[pl.ds(ubase, U_PER_W)])

        vbase = wid * V_PER_W

        def body(i, carry):
            b = vbase + i * V_CHUNK
            pltpu.sync_copy(idxv_h.at[pl.ds(b, V_CHUNK)], idxv_buf)
            pltpu.async_copy(emb_h.at[idxv_buf], vrow_buf, sem).wait()
            pltpu.sync_copy(vrow_buf, to_h.at[pl.ds(b, V_CHUNK)])
            return carry

        lax.fori_loop(0, V_STEPS, body, 0)

    return gather_kernel(emb, idx_u, idx_v)


def _distance_body(t_sel_ref, g_sum_ref, from_ref, to_ref, out_ref):
    u = from_ref[...]
    t = to_ref[...]
    t_sel = t_sel_ref[...]
    g_sum = g_sum_ref[...]
    hi = jax.lax.Precision.HIGHEST
    u_rep = jnp.dot(u, t_sel, precision=hi,
                    preferred_element_type=jnp.float32)
    d = t - u_rep
    sqdist = jnp.dot(d * d, g_sum, precision=hi,
                     preferred_element_type=jnp.float32)
    sqvnorm = jnp.dot(t * t, g_sum, precision=hi,
                      preferred_element_type=jnp.float32)
    squnorm = jnp.sum(u * u, axis=-1, keepdims=True)
    squnorm = jnp.clip(squnorm, 0.0, 1.0 - EPS)
    sqvnorm = jnp.clip(sqvnorm, 0.0, 1.0 - EPS)
    x = 1.0 + 2.0 * sqdist / ((1.0 - squnorm) * (1.0 - sqvnorm))
    x = jnp.maximum(x, 1.0 + EPS)
    out_ref[...] = -jnp.log(x + jnp.sqrt((x + 1.0) * (x - 1.0)))


def _tc_distance(from_rows, to_flat):
    bb = 2048
    grid = BATCH // bb
    t_sel = jnp.tile(jnp.eye(DIM, dtype=jnp.float32), (1, NNEG))
    g_sum = jnp.repeat(jnp.eye(NNEG, dtype=jnp.float32), DIM, axis=0)
    return pl.pallas_call(
        _distance_body,
        grid=(grid,),
        in_specs=[
            pl.BlockSpec((DIM, NNEG * DIM), lambda i: (0, 0)),
            pl.BlockSpec((NNEG * DIM, NNEG), lambda i: (0, 0)),
            pl.BlockSpec((bb, DIM), lambda i: (i, 0)),
            pl.BlockSpec((bb, NNEG * DIM), lambda i: (i, 0)),
        ],
        out_specs=pl.BlockSpec((bb, NNEG), lambda i: (i, 0)),
        out_shape=jax.ShapeDtypeStruct((BATCH, NNEG), jnp.float32),
    )(t_sel, g_sum, from_rows, to_flat)


def kernel(inputs, emb):
    idx_u = inputs[:, 0].astype(jnp.int32)
    idx_v = inputs[:, 1:].reshape(-1).astype(jnp.int32)
    from_rows, to_rows = _sc_gather(emb, idx_u, idx_v)
    to_flat = to_rows.reshape(BATCH, NNEG * DIM)
    return _tc_distance(from_rows, to_flat)


# speedup vs baseline: 1.3961x; 1.1767x over previous
import functools

import jax
import jax.numpy as jnp
from jax import lax
from jax.experimental import pallas as pl
from jax.experimental.pallas import tpu as pltpu
from jax.experimental.pallas import tpu_sc as plsc

SIZE = 1000000
DIM = 16
BATCH = 16384
NCOLS = 51
NNEG = NCOLS - 1
EPS = 1e-10

NC = 2
NS = 16
NW = NC * NS
LANES = 16

B_PER_W = BATCH // NW
CB = 64
N_CHUNKS = B_PER_W // CB
ROWS_PER_CHUNK = CB * NCOLS

_MAGIC = 0x5F3759DF


def _sqrt_pos(a):
    bits = plsc.bitcast(a, jnp.int32)
    r = plsc.bitcast(_MAGIC - lax.shift_right_logical(bits, 1), jnp.float32)
    half_a = 0.5 * a
    for _ in range(3):
        r = r * (1.5 - half_a * r * r)
    return a * r


def _dist_chunk(rows_buf, x_buf):
    iota = lax.iota(jnp.int32, LANES)
    col = jnp.zeros((LANES,), jnp.int32)

    def group_body(g, _g):
        b16 = g * LANES + iota
        urow = b16 * NCOLS
        u_d = []
        un = jnp.zeros((LANES,), jnp.float32)
        for d in range(DIM):
            ud = plsc.load_gather(rows_buf, [urow, col + d])
            u_d.append(ud)
            un = un + ud * ud

        def j_body(j, _j):
            vrow = urow + 1 + j
            sqd = jnp.zeros((LANES,), jnp.float32)
            vn = jnp.zeros((LANES,), jnp.float32)
            for d in range(DIM):
                vd = plsc.load_gather(rows_buf, [vrow, col + d])
                diff = vd - u_d[d]
                sqd = sqd + diff * diff
                vn = vn + vd * vd
            un_c = jnp.minimum(un, 1.0 - EPS)
            vn_c = jnp.minimum(vn, 1.0 - EPS)
            t = 2.0 * sqd / ((1.0 - un_c) * (1.0 - vn_c))
            x = jnp.maximum(1.0 + t, 1.0 + EPS)
            t2 = x - 1.0
            t2 = jnp.maximum(t2, 1e-30)
            w = t2 + _sqrt_pos((2.0 + t2) * t2)
            z = w / (2.0 + w)
            acosh = 2.0 * z + 0.666666667 * z * z * z
            plsc.store_scatter(x_buf, [b16, col + j], -acosh)
            return _j

        lax.fori_loop(0, NNEG, j_body, 0)
        return _g

    lax.fori_loop(0, CB // LANES, group_body, 0)


def _sc_kernel_fn(emb_h, idx_h, out_h, idx_buf, rows_buf, x_buf, sem):
    wid = lax.axis_index("s") * NC + lax.axis_index("c")
    b0 = wid * B_PER_W

    def chunk_body(ci, carry):
        b = b0 + ci * CB
        pltpu.sync_copy(idx_h.at[pl.ds(b * NCOLS, ROWS_PER_CHUNK)], idx_buf)
        pltpu.async_copy(emb_h.at[idx_buf], rows_buf, sem).wait()
        _dist_chunk(rows_buf, x_buf)
        pltpu.sync_copy(x_buf, out_h.at[pl.ds(b, CB)])
        return carry

    lax.fori_loop(0, N_CHUNKS, chunk_body, 0)


def kernel(inputs, emb):
    idx_flat = inputs.reshape(-1).astype(jnp.int32)
    mesh = plsc.VectorSubcoreMesh(core_axis_name="c", subcore_axis_name="s")
    sc = functools.partial(
        pl.kernel,
        out_type=jax.ShapeDtypeStruct((BATCH, NNEG), jnp.float32),
        mesh=mesh,
        compiler_params=pltpu.CompilerParams(
            use_tc_tiling_on_sc=False, needs_layout_passes=False),
        scratch_types=(
            pltpu.VMEM((ROWS_PER_CHUNK,), jnp.int32),
            pltpu.VMEM((ROWS_PER_CHUNK, DIM), jnp.float32),
            pltpu.VMEM((CB, NNEG), jnp.float32),
            pltpu.SemaphoreType.DMA,
        ),
    )(_sc_kernel_fn)
    return sc(emb, idx_flat)


# speedup vs baseline: 1.4381x; 1.0301x over previous
import functools

import jax
import jax.numpy as jnp
from jax import lax
from jax.experimental import pallas as pl
from jax.experimental.pallas import tpu as pltpu
from jax.experimental.pallas import tpu_sc as plsc

SIZE = 1000000
DIM = 16
BATCH = 16384
NCOLS = 51
NNEG = NCOLS - 1
EPS = 1e-10

NC = 2
NS = 16
NW = NC * NS
LANES = 16

B_PER_W = BATCH // NW
CB = 32
N_CHUNKS = B_PER_W // CB
ROWS_PER_CHUNK = CB * NCOLS
STREAM = 96
N_STREAMS = ROWS_PER_CHUNK // STREAM

_MAGIC = 0x5F3759DF


def _sqrt_pos(a):
    bits = plsc.bitcast(a, jnp.int32)
    r = plsc.bitcast(_MAGIC - lax.shift_right_logical(bits, 1), jnp.float32)
    half_a = 0.5 * a
    for _ in range(3):
        r = r * (1.5 - half_a * r * r)
    return a * r


def _dist_chunk(rows_buf, x_buf):
    iota = lax.iota(jnp.int32, LANES)
    zero = jnp.zeros((LANES,), jnp.int32)

    def group_body(g, _g):
        lb16 = g * LANES + iota
        b16 = lb16 * NCOLS
        u_d = []
        un = jnp.zeros((LANES,), jnp.float32)
        for d in range(DIM):
            ud = plsc.load_gather(rows_buf, [b16, zero + d])
            u_d.append(ud)
            un = un + ud * ud

        def j_body(j, _j):
            row = b16 + (1 + j)
            sqd = jnp.zeros((LANES,), jnp.float32)
            vn = jnp.zeros((LANES,), jnp.float32)
            for d in range(DIM):
                vd = plsc.load_gather(rows_buf, [row, zero + d])
                diff = vd - u_d[d]
                sqd = sqd + diff * diff
                vn = vn + vd * vd
            un_c = jnp.minimum(un, 1.0 - EPS)
            vn_c = jnp.minimum(vn, 1.0 - EPS)
            t = 2.0 * sqd / ((1.0 - un_c) * (1.0 - vn_c))
            x = jnp.maximum(1.0 + t, 1.0 + EPS)
            t2 = x - 1.0
            t2 = jnp.maximum(t2, 1e-30)
            w = t2 + _sqrt_pos((2.0 + t2) * t2)
            z = w / (2.0 + w)
            acosh = 2.0 * z + 0.666666667 * z * z * z
            plsc.store_scatter(x_buf, [lb16, zero + j], -acosh)
            return _j

        lax.fori_loop(0, NNEG, j_body, 0)
        return _g

    lax.fori_loop(0, CB // LANES, group_body, 0)


def _sc_kernel_fn(emb_h, idx_h, out_h,
                  idx0, idx1, rows0, rows1, x_buf, sem0, sem1):
    wid = lax.axis_index("s") * NC + lax.axis_index("c")
    b0 = wid * B_PER_W
    idx_bufs = (idx0, idx1)
    rows_bufs = (rows0, rows1)
    sems = (sem0, sem1)

    def fetch(ci, slot):
        base = (b0 + ci * CB) * NCOLS
        pltpu.sync_copy(idx_h.at[pl.ds(base, ROWS_PER_CHUNK)], idx_bufs[slot])
        cps = []
        for k in range(N_STREAMS):
            sl = pl.ds(k * STREAM, STREAM)
            cps.append(pltpu.async_copy(
                emb_h.at[idx_bufs[slot].at[sl]], rows_bufs[slot].at[sl],
                sems[slot]))
        return cps

    cps = fetch(0, 0)
    for ci in range(N_CHUNKS):
        slot = ci % 2
        for cp in cps:
            cp.wait()
        if ci + 1 < N_CHUNKS:
            cps = fetch(ci + 1, 1 - slot)
        _dist_chunk(rows_bufs[slot], x_buf)
        pltpu.sync_copy(x_buf, out_h.at[pl.ds(b0 + ci * CB, CB)])


def kernel(inputs, emb):
    idx_flat = inputs.reshape(-1).astype(jnp.int32)
    mesh = plsc.VectorSubcoreMesh(core_axis_name="c", subcore_axis_name="s")
    sc = functools.partial(
        pl.kernel,
        out_type=jax.ShapeDtypeStruct((BATCH, NNEG), jnp.float32),
        mesh=mesh,
        compiler_params=pltpu.CompilerParams(
            use_tc_tiling_on_sc=False, needs_layout_passes=False),
        scratch_types=(
            pltpu.VMEM((ROWS_PER_CHUNK,), jnp.int32),
            pltpu.VMEM((ROWS_PER_CHUNK,), jnp.int32),
            pltpu.VMEM((ROWS_PER_CHUNK, DIM), jnp.float32),
            pltpu.VMEM((ROWS_PER_CHUNK, DIM), jnp.float32),
            pltpu.VMEM((CB, NNEG), jnp.float32),
            pltpu.SemaphoreType.DMA,
            pltpu.SemaphoreType.DMA,
        ),
    )(_sc_kernel_fn)
    return sc(emb, idx_flat)
